# baseline (speedup 1.0000x reference)
import functools

import jax
import jax.numpy as jnp
from jax import lax
from jax.experimental import pallas as pl
from jax.experimental.pallas import tpu as pltpu
from jax.experimental.pallas import tpu_sc as plsc

N = 50000
E = 800000
D_IN = 100
D = 64
NG = 512
N_OUT = 24

NC = 2
NS = 16
HALF = N // NC
ROWS_PT = 1568
ROWS_LAST = HALF - (NS - 1) * ROWS_PT
ACC_ROWS = NS * ROWS_PT
EDGE_K = 80
EPT = E // NS
STEPS = EPT // EDGE_K
ZROWS = 112

POOL_K = 112
POOL_FULL = ROWS_PT // POOL_K
POOL_LAST_FULL = ROWS_LAST // POOL_K
POOL_TAIL = ROWS_LAST - POOL_LAST_FULL * POOL_K
GPT = NG // NS


def _sc_mesh():
    return plsc.VectorSubcoreMesh(
        core_axis_name="c", subcore_axis_name="s", num_cores=NC, num_subcores=NS
    )


def _zero_vmem(buf, rows):
    z16 = jnp.zeros((16,), jnp.float32)

    def zrow(i, _):
        for j in range(D // 16):
            buf[i, pl.ds(j * 16, 16)] = z16
        return 0

    lax.fori_loop(0, rows, zrow, 0)


def _edge_segsum(p, src, dst):
    @functools.partial(
        pl.kernel,
        out_type=jax.ShapeDtypeStruct((N, D), jnp.float32),
        mesh=_sc_mesh(),
        compiler_params=pltpu.CompilerParams(use_tc_tiling_on_sc=False),
        scratch_types=[
            pltpu.VMEM((EDGE_K,), jnp.int32),
            pltpu.VMEM((EDGE_K,), jnp.int32),
            pltpu.VMEM((EDGE_K,), jnp.int32),
            pltpu.VMEM((EDGE_K,), jnp.int32),
            pltpu.VMEM((EDGE_K,), jnp.int32),
            pltpu.VMEM((EDGE_K,), jnp.int32),
            pltpu.VMEM((EDGE_K, D), jnp.float32),
            pltpu.VMEM((EDGE_K, D), jnp.float32),
            pltpu.VMEM((ZROWS, D), jnp.float32),
            pltpu.VMEM_SHARED((ACC_ROWS, D), jnp.float32),
            pltpu.SemaphoreType.DMA,
            pltpu.SemaphoreType.DMA,
            pltpu.SemaphoreType.DMA,
            pltpu.SemaphoreType.DMA,
            pltpu.SemaphoreType.DMA,
            pltpu.SemaphoreType.DMA,
        ],
    )
    def k(p_hbm, src_hbm, dst_hbm, out_hbm, sbuf0, sbuf1, dbuf0, dbuf1,
          dloc0, dloc1, rows0, rows1, zbuf, acc, semi0, semi1, semg0, semg1,
          sems0, sems1):
        c = lax.axis_index("c")
        s = lax.axis_index("s")
        base = c * HALF

        _zero_vmem(zbuf, ZROWS)

        def zcopy(kk, _):
            pltpu.sync_copy(zbuf, acc.at[pl.ds(s * ROWS_PT + kk * ZROWS, ZROWS)])
            return 0

        lax.fori_loop(0, ROWS_PT // ZROWS, zcopy, 0)
        plsc.subcore_barrier()

        lanes = lax.iota(jnp.int32, 16)
        estart = s * EPT

        def idx_load(m, sbuf, dbuf, semi):
            eb = estart + jnp.where(m < STEPS, m, 0) * EDGE_K
            pltpu.async_copy(src_hbm.at[pl.ds(eb, EDGE_K)], sbuf, semi)
            pltpu.async_copy(dst_hbm.at[pl.ds(eb, EDGE_K)], dbuf, semi)

        def idx_wait(sbuf, dbuf, semi):
            pltpu.make_async_copy(src_hbm.at[pl.ds(0, EDGE_K)], sbuf, semi).wait()
            pltpu.make_async_copy(dst_hbm.at[pl.ds(0, EDGE_K)], dbuf, semi).wait()

        def fire_gather(sbuf, rows, semg):
            pltpu.async_copy(p_hbm.at[sbuf], rows, semg)

        def gather_wait(rows, semg):
            pltpu.make_async_copy(p_hbm.at[pl.ds(0, EDGE_K)], rows, semg).wait()

        def scatter_wait(rows, sems):
            pltpu.make_async_copy(p_hbm.at[pl.ds(0, EDGE_K)], rows, sems).wait()

        def compute_dloc(dbuf, dloc):
            for kk in range(EDGE_K // 16):
                d = dbuf[pl.ds(kk * 16, 16)]
                inr = (d >= base) & (d < base + HALF)
                dummy = HALF + kk * 16 + lanes
                dloc[pl.ds(kk * 16, 16)] = jnp.where(inr, d - base, dummy)

        def macro(m, drain, cur, nxt):
            (sbuf_c, dbuf_c, dloc_c, rows_c, semi_c, semg_c, sems_c) = cur
            (sbuf_n, dbuf_n, rows_n, semi_n, semg_n, sems_n) = nxt
            idx_wait(sbuf_n, dbuf_n, semi_n)
            if drain:
                scatter_wait(rows_n, sems_n)
            fire_gather(sbuf_n, rows_n, semg_n)
            compute_dloc(dbuf_c, dloc_c)
            gather_wait(rows_c, semg_c)
            idx_load(m + 2, sbuf_c, dbuf_c, semi_c)
            pltpu.async_copy(rows_c, acc.at[dloc_c], sems_c, add=True)

        bufs0 = (sbuf0, dbuf0, dloc0, rows0, semi0, semg0, sems0)
        bufs1 = (sbuf1, dbuf1, dloc1, rows1, semi1, semg1, sems1)
        nxt0 = (sbuf0, dbuf0, rows0, semi0, semg0, sems0)
        nxt1 = (sbuf1, dbuf1, rows1, semi1, semg1, sems1)

        idx_load(jnp.int32(0), sbuf0, dbuf0, semi0)
        idx_load(jnp.int32(1), sbuf1, dbuf1, semi1)
        idx_wait(sbuf0, dbuf0, semi0)
        fire_gather(sbuf0, rows0, semg0)

        macro(jnp.int32(0), False, bufs0, nxt1)
        macro(jnp.int32(1), True, bufs1, nxt0)

        def pair(i, _):
            m = 2 * i
            macro(m, True, bufs0, nxt1)
            macro(m + 1, True, bufs1, nxt0)
            return 0

        lax.fori_loop(1, STEPS // 2, pair, 0)

        scatter_wait(rows1, sems1)
        compute_dloc(dbuf0, dloc0)
        gather_wait(rows0, semg0)
        pltpu.async_copy(rows0, acc.at[dloc0], sems0, add=True).wait()
        idx_wait(sbuf1, dbuf1, semi1)
        plsc.subcore_barrier()

        @pl.when(s < NS - 1)
        def _():
            pltpu.sync_copy(
                acc.at[pl.ds(s * ROWS_PT, ROWS_PT)],
                out_hbm.at[pl.ds(base + s * ROWS_PT, ROWS_PT)],
            )

        @pl.when(s == NS - 1)
        def _():
            pltpu.sync_copy(
                acc.at[pl.ds((NS - 1) * ROWS_PT, ROWS_LAST)],
                out_hbm.at[pl.ds(base + (NS - 1) * ROWS_PT, ROWS_LAST)],
            )

    return k(p, src, dst)


def _pool_segsum(sarr, batch):
    @functools.partial(
        pl.kernel,
        out_type=jax.ShapeDtypeStruct((NC * NG, D), jnp.float32),
        mesh=_sc_mesh(),
        compiler_params=pltpu.CompilerParams(use_tc_tiling_on_sc=False),
        scratch_types=[
            pltpu.VMEM((POOL_K,), jnp.int32),
            pltpu.VMEM((POOL_TAIL,), jnp.int32),
            pltpu.VMEM((POOL_K, D), jnp.float32),
            pltpu.VMEM((GPT, D), jnp.float32),
            pltpu.VMEM_SHARED((NG, D), jnp.float32),
        ],
    )
    def k(s_hbm, b_hbm, out_hbm, bbuf, bbuf_t, rows, zbuf, acc):
        c = lax.axis_index("c")
        s = lax.axis_index("s")

        _zero_vmem(zbuf, GPT)
        pltpu.sync_copy(zbuf, acc.at[pl.ds(s * GPT, GPT)])
        plsc.subcore_barrier()

        rbase = c * HALF + s * ROWS_PT
        nfull = jnp.where(s < NS - 1, POOL_FULL, POOL_LAST_FULL)

        def step(kk, _):
            rb = rbase + kk * POOL_K
            pltpu.sync_copy(s_hbm.at[pl.ds(rb, POOL_K)], rows)
            pltpu.sync_copy(b_hbm.at[pl.ds(rb, POOL_K)], bbuf)
            pltpu.sync_copy(rows, acc.at[bbuf], add=True)
            return 0

        lax.fori_loop(0, nfull, step, 0)

        @pl.when(s == NS - 1)
        def _():
            rb = rbase + POOL_LAST_FULL * POOL_K
            pltpu.sync_copy(s_hbm.at[pl.ds(rb, POOL_TAIL)], rows.at[pl.ds(0, POOL_TAIL)])
            pltpu.sync_copy(b_hbm.at[pl.ds(rb, POOL_TAIL)], bbuf_t)
            pltpu.sync_copy(rows.at[pl.ds(0, POOL_TAIL)], acc.at[bbuf_t], add=True)

        plsc.subcore_barrier()
        pltpu.sync_copy(
            acc.at[pl.ds(s * GPT, GPT)],
            out_hbm.at[pl.ds(c * NG + s * GPT, GPT)],
        )

    return k(sarr, batch)


BR = 2000


def _mm_in(x, w1):
    def body(x_ref, w_ref, o_ref):
        o_ref[...] = jnp.dot(x_ref[...], w_ref[...], preferred_element_type=jnp.float32)

    return pl.pallas_call(
        body,
        grid=(N // BR,),
        in_specs=[
            pl.BlockSpec((BR, D_IN), lambda i: (i, 0)),
            pl.BlockSpec((D_IN, D), lambda i: (0, 0)),
        ],
        out_specs=pl.BlockSpec((BR, D), lambda i: (i, 0)),
        out_shape=jax.ShapeDtypeStruct((N, D), jnp.float32),
    )(x, w1)


def _layer_mid(p, agg, part, eps, b1, w2, b2, wnext, jkw):
    def body(eps_ref, p_ref, a_ref, pt_ref, b1_ref, w2_ref, b2_ref, wn_ref,
             jk_ref, pn_ref, po_ref):
        z = (1.0 + eps_ref[0]) * p_ref[...] + a_ref[...] + b1_ref[...]
        r = jnp.maximum(z, 0.0)
        h = jnp.maximum(
            jnp.dot(r, w2_ref[...], preferred_element_type=jnp.float32) + b2_ref[...],
            0.0,
        )
        pn_ref[...] = jnp.dot(h, wn_ref[...], preferred_element_type=jnp.float32)
        po_ref[...] = pt_ref[...] + jnp.dot(
            h, jk_ref[...], preferred_element_type=jnp.float32
        )

    full = lambda shape: pl.BlockSpec(shape, lambda i: tuple(0 for _ in shape))
    return pl.pallas_call(
        body,
        grid=(N // BR,),
        in_specs=[
            pl.BlockSpec(memory_space=pltpu.SMEM),
            pl.BlockSpec((BR, D), lambda i: (i, 0)),
            pl.BlockSpec((BR, D), lambda i: (i, 0)),
            pl.BlockSpec((BR, D), lambda i: (i, 0)),
            full((1, D)),
            full((D, D)),
            full((1, D)),
            full((D, D)),
            full((D, D)),
        ],
        out_specs=[
            pl.BlockSpec((BR, D), lambda i: (i, 0)),
            pl.BlockSpec((BR, D), lambda i: (i, 0)),
        ],
        out_shape=[
            jax.ShapeDtypeStruct((N, D), jnp.float32),
            jax.ShapeDtypeStruct((N, D), jnp.float32),
        ],
    )(eps.reshape(1), p, agg, part, b1.reshape(1, D), w2, b2.reshape(1, D), wnext, jkw)


def _layer_first(p, agg, eps, b1, w2, b2, wnext, jkw):
    def body(eps_ref, p_ref, a_ref, b1_ref, w2_ref, b2_ref, wn_ref, jk_ref,
             pn_ref, po_ref):
        z = (1.0 + eps_ref[0]) * p_ref[...] + a_ref[...] + b1_ref[...]
        r = jnp.maximum(z, 0.0)
        h = jnp.maximum(
            jnp.dot(r, w2_ref[...], preferred_element_type=jnp.float32) + b2_ref[...],
            0.0,
        )
        pn_ref[...] = jnp.dot(h, wn_ref[...], preferred_element_type=jnp.float32)
        po_ref[...] = jnp.dot(h, jk_ref[...], preferred_element_type=jnp.float32)

    full = lambda shape: pl.BlockSpec(shape, lambda i: tuple(0 for _ in shape))
    return pl.pallas_call(
        body,
        grid=(N // BR,),
        in_specs=[
            pl.BlockSpec(memory_space=pltpu.SMEM),
            pl.BlockSpec((BR, D), lambda i: (i, 0)),
            pl.BlockSpec((BR, D), lambda i: (i, 0)),
            full((1, D)),
            full((D, D)),
            full((1, D)),
            full((D, D)),
            full((D, D)),
        ],
        out_specs=[
            pl.BlockSpec((BR, D), lambda i: (i, 0)),
            pl.BlockSpec((BR, D), lambda i: (i, 0)),
        ],
        out_shape=[
            jax.ShapeDtypeStruct((N, D), jnp.float32),
            jax.ShapeDtypeStruct((N, D), jnp.float32),
        ],
    )(eps.reshape(1), p, agg, b1.reshape(1, D), w2, b2.reshape(1, D), wnext, jkw)


def _layer_last(p, agg, part, eps, b1, w2, b2, jkw, jkb):
    def body(eps_ref, p_ref, a_ref, pt_ref, b1_ref, w2_ref, b2_ref, jk_ref,
             jkb_ref, so_ref):
        z = (1.0 + eps_ref[0]) * p_ref[...] + a_ref[...] + b1_ref[...]
        r = jnp.maximum(z, 0.0)
        h = jnp.maximum(
            jnp.dot(r, w2_ref[...], preferred_element_type=jnp.float32) + b2_ref[...],
            0.0,
        )
        so_ref[...] = (
            pt_ref[...]
            + jnp.dot(h, jk_ref[...], preferred_element_type=jnp.float32)
            + jkb_ref[...]
        )

    full = lambda shape: pl.BlockSpec(shape, lambda i: tuple(0 for _ in shape))
    return pl.pallas_call(
        body,
        grid=(N // BR,),
        in_specs=[
            pl.BlockSpec(memory_space=pltpu.SMEM),
            pl.BlockSpec((BR, D), lambda i: (i, 0)),
            pl.BlockSpec((BR, D), lambda i: (i, 0)),
            pl.BlockSpec((BR, D), lambda i: (i, 0)),
            full((1, D)),
            full((D, D)),
            full((1, D)),
            full((D, D)),
            full((1, D)),
        ],
        out_specs=pl.BlockSpec((BR, D), lambda i: (i, 0)),
        out_shape=jax.ShapeDtypeStruct((N, D), jnp.float32),
    )(eps.reshape(1), p, agg, part, b1.reshape(1, D), w2, b2.reshape(1, D), jkw,
      jkb.reshape(1, D))


def _head(gparts, ffn_w1, ffn_b1, bn_g, bn_b, bn_m, bn_v, ffn_w2, ffn_b2,
          out_w, out_b):
    def body(gp_ref, w1_ref, b1_ref, g_ref, bb_ref, m_ref, v_ref, w2_ref,
             b2_ref, ow_ref, ob_ref, o_ref):
        g = gp_ref[:NG, :] + gp_ref[NG:, :]
        f = jnp.dot(g, w1_ref[...], preferred_element_type=jnp.float32) + b1_ref[...]
        scale = g_ref[...] * lax.rsqrt(v_ref[...] + 1e-5)
        f = (f - m_ref[...]) * scale + bb_ref[...]
        f = jnp.maximum(f, 0.0)
        f = jnp.dot(f, w2_ref[...], preferred_element_type=jnp.float32) + b2_ref[...]
        o_ref[...] = (
            jnp.dot(f, ow_ref[...], preferred_element_type=jnp.float32) + ob_ref[...]
        )

    full = lambda shape: pl.BlockSpec(shape, lambda: tuple(0 for _ in shape))
    return pl.pallas_call(
        body,
        in_specs=[
            full((NC * NG, D)),
            full((D, D)),
            full((1, D)),
            full((1, D)),
            full((1, D)),
            full((1, D)),
            full((1, D)),
            full((D, D)),
            full((1, D)),
            full((D, N_OUT)),
            full((1, N_OUT)),
        ],
        out_specs=full((NG, N_OUT)),
        out_shape=jax.ShapeDtypeStruct((NG, N_OUT), jnp.float32),
    )(gparts, ffn_w1, ffn_b1.reshape(1, D), bn_g.reshape(1, D),
      bn_b.reshape(1, D), bn_m.reshape(1, D), bn_v.reshape(1, D), ffn_w2,
      ffn_b2.reshape(1, D), out_w, out_b.reshape(1, N_OUT))


def kernel(x, edge_index, batch, eps0, l0_W1, l0_b1, l0_W2, l0_b2, eps1,
           l1_W1, l1_b1, l1_W2, l1_b2, eps2, l2_W1, l2_b1, l2_W2, l2_b2,
           jk_W, jk_b, ffn_W1, ffn_b1, bn_g, bn_b, bn_m, bn_v, ffn_W2,
           ffn_b2, out_W, out_b):
    src = edge_index[0]
    dst = edge_index[1]
    jkw0, jkw1, jkw2 = jk_W[:D], jk_W[D:2 * D], jk_W[2 * D:]

    p0 = _mm_in(x, l0_W1)
    a0 = _edge_segsum(p0, src, dst)
    p1, part = _layer_first(p0, a0, eps0, l0_b1, l0_W2, l0_b2, l1_W1, jkw0)
    a1 = _edge_segsum(p1, src, dst)
    p2, part = _layer_mid(p1, a1, part, eps1, l1_b1, l1_W2, l1_b2, l2_W1, jkw1)
    a2 = _edge_segsum(p2, src, dst)
    sarr = _layer_last(p2, a2, part, eps2, l2_b1, l2_W2, l2_b2, jkw2, jk_b)
    gparts = _pool_segsum(sarr, batch)
    return _head(gparts, ffn_W1, ffn_b1, bn_g, bn_b, bn_m, bn_v, ffn_W2,
                 ffn_b2, out_W, out_b)


# speedup vs baseline: 1.3607x; 1.3607x over previous
import functools

import jax
import jax.numpy as jnp
from jax import lax
from jax.experimental import pallas as pl
from jax.experimental.pallas import tpu as pltpu
from jax.experimental.pallas import tpu_sc as plsc

N = 50000
E = 800000
D_IN = 100
D = 64
NG = 512
N_OUT = 24

NC = 2
NS = 16
HALF = N // NC
ROWS_PT = 1568
ROWS_LAST = HALF - (NS - 1) * ROWS_PT
ACC_ROWS = NS * ROWS_PT
EDGE_K = 80
EPT = E // NS
STEPS = EPT // EDGE_K
ZROWS = 112

CAP = 50480
FLUSH = 2048
STG = FLUSH + 400
PBLK = 2000

POOL_K = 112
POOL_FULL = ROWS_PT // POOL_K
POOL_LAST_FULL = ROWS_LAST // POOL_K
POOL_TAIL = ROWS_LAST - POOL_LAST_FULL * POOL_K
GPT = NG // NS


def _sc_mesh():
    return plsc.VectorSubcoreMesh(
        core_axis_name="c", subcore_axis_name="s", num_cores=NC, num_subcores=NS
    )


def _zero_vmem(buf, rows):
    z16 = jnp.zeros((16,), jnp.float32)

    def zrow(i, _):
        for j in range(D // 16):
            buf[i, pl.ds(j * 16, 16)] = z16
        return 0

    lax.fori_loop(0, rows, zrow, 0)


def _compact_edges(src, dst):
    @functools.partial(
        pl.kernel,
        out_type=(
            jax.ShapeDtypeStruct((NC, NS, CAP), jnp.int32),
            jax.ShapeDtypeStruct((NC, NS, CAP), jnp.int32),
            jax.ShapeDtypeStruct((NC, NS, 16), jnp.int32),
        ),
        mesh=_sc_mesh(),
        compiler_params=pltpu.CompilerParams(use_tc_tiling_on_sc=False,
                                             needs_layout_passes=False),
        scratch_types=[
            pltpu.VMEM((PBLK,), jnp.int32),
            pltpu.VMEM((PBLK,), jnp.int32),
            pltpu.VMEM((STG,), jnp.int32),
            pltpu.VMEM((STG,), jnp.int32),
            pltpu.VMEM((16,), jnp.int32),
        ],
    )
    def k(src_hbm, dst_hbm, plist, dlist, cnts, sblk, dblk, pstg, dstg, cbuf):
        c = lax.axis_index("c")
        s = lax.axis_index("s")
        base = c * HALF
        lanes = lax.iota(jnp.int32, 16)
        estart = s * EPT

        def block(bi, carry):
            off0, fl0 = carry
            eb = estart + bi * PBLK
            pltpu.sync_copy(src_hbm.at[pl.ds(eb, PBLK)], sblk)
            pltpu.sync_copy(dst_hbm.at[pl.ds(eb, PBLK)], dblk)

            def chunk(kk, carry2):
                off, fl = carry2
                d = dblk[pl.ds(kk * 16, 16)]
                sv = sblk[pl.ds(kk * 16, 16)]
                inr = (d >= base) & (d < base + HALF)
                mi = jnp.where(inr, jnp.int32(1), jnp.int32(0))
                incl = plsc.cumsum(mi)
                idx = off + (incl - mi)
                plsc.store_scatter(pstg, [idx], sv, mask=inr)
                plsc.store_scatter(dstg, [idx], d - base, mask=inr)
                off = off + jnp.max(incl)
                do_flush = off >= FLUSH

                @pl.when(do_flush)
                def _():
                    fla = pl.multiple_of(fl, 8)
                    pltpu.sync_copy(pstg.at[pl.ds(0, FLUSH)],
                                    plist.at[c, s, pl.ds(fla, FLUSH)])
                    pltpu.sync_copy(dstg.at[pl.ds(0, FLUSH)],
                                    dlist.at[c, s, pl.ds(fla, FLUSH)])
                    pstg[pl.ds(0, 16)] = pstg[pl.ds(FLUSH, 16)]
                    dstg[pl.ds(0, 16)] = dstg[pl.ds(FLUSH, 16)]

                off = jnp.where(do_flush, off - FLUSH, off)
                fl = jnp.where(do_flush, fl + FLUSH, fl)
                return (off, fl)

            return lax.fori_loop(0, PBLK // 16, chunk, (off0, fl0))

        off, fl = lax.fori_loop(0, EPT // PBLK, block,
                                (jnp.int32(0), jnp.int32(0)))

        t0 = fl + off
        p1 = lax.rem(jnp.int32(80) - lax.rem(t0, 160) + 160, 160)
        padn = p1 + jnp.where(t0 + p1 < 240, jnp.int32(160), jnp.int32(0))
        for j in range(25):
            pstg[pl.ds(off + j * 16, 16)] = jnp.zeros((16,), jnp.int32)
            dstg[pl.ds(off + j * 16, 16)] = HALF + (j % 5) * 16 + lanes
        off = off + padn
        t = t0 + padn

        def fflush(j, _):
            fo = pl.multiple_of(fl + j * 80, 8)
            pltpu.sync_copy(pstg.at[pl.ds(j * 80, 80)],
                            plist.at[c, s, pl.ds(fo, 80)])
            pltpu.sync_copy(dstg.at[pl.ds(j * 80, 80)],
                            dlist.at[c, s, pl.ds(fo, 80)])
            return 0

        lax.fori_loop(0, (off + 79) // 80, fflush, 0)
        cbuf[pl.ds(0, 16)] = jnp.zeros((16,), jnp.int32) + t
        pltpu.sync_copy(cbuf, cnts.at[c, s])

    return k(src, dst)


def _edge_segsum(p, plist, dlist, cnts):
    @functools.partial(
        pl.kernel,
        out_type=jax.ShapeDtypeStruct((N, D), jnp.float32),
        mesh=_sc_mesh(),
        compiler_params=pltpu.CompilerParams(use_tc_tiling_on_sc=False,
                                             needs_layout_passes=False),
        scratch_types=[
            pltpu.VMEM((EDGE_K,), jnp.int32),
            pltpu.VMEM((EDGE_K,), jnp.int32),
            pltpu.VMEM((EDGE_K,), jnp.int32),
            pltpu.VMEM((EDGE_K,), jnp.int32),
            pltpu.VMEM((EDGE_K,), jnp.int32),
            pltpu.VMEM((EDGE_K,), jnp.int32),
            pltpu.VMEM((EDGE_K, D), jnp.float32),
            pltpu.VMEM((EDGE_K, D), jnp.float32),
            pltpu.VMEM((ZROWS, D), jnp.float32),
            pltpu.VMEM((16,), jnp.int32),
            pltpu.VMEM_SHARED((ACC_ROWS, D), jnp.float32),
            pltpu.SemaphoreType.DMA,
            pltpu.SemaphoreType.DMA,
            pltpu.SemaphoreType.DMA,
            pltpu.SemaphoreType.DMA,
            pltpu.SemaphoreType.DMA,
            pltpu.SemaphoreType.DMA,
        ],
    )
    def k(p_hbm, pl_hbm, dl_hbm, cn_hbm, out_hbm, sbuf0, sbuf1, dbuf0, dbuf1,
          dloc0, dloc1, rows0, rows1, zbuf, cbuf, acc, semi0, semi1, semg0,
          semg1, sems0, sems1):
        c = lax.axis_index("c")
        s = lax.axis_index("s")
        base = c * HALF

        _zero_vmem(zbuf, ZROWS)

        def zcopy(kk, _):
            pltpu.sync_copy(zbuf, acc.at[pl.ds(s * ROWS_PT + kk * ZROWS, ZROWS)])
            return 0

        lax.fori_loop(0, ROWS_PT // ZROWS, zcopy, 0)
        plsc.subcore_barrier()

        pltpu.sync_copy(cn_hbm.at[c, s], cbuf)
        trips = jnp.max(cbuf[...]) // EDGE_K

        def idx_load(m, sbuf, dbuf, semi):
            eb = pl.multiple_of(jnp.where(m < trips, m, 0) * EDGE_K, 8)
            pltpu.async_copy(pl_hbm.at[c, s, pl.ds(eb, EDGE_K)], sbuf, semi)
            pltpu.async_copy(dl_hbm.at[c, s, pl.ds(eb, EDGE_K)], dbuf, semi)

        def idx_wait(sbuf, dbuf, semi):
            pltpu.make_async_copy(pl_hbm.at[c, s, pl.ds(0, EDGE_K)], sbuf,
                                  semi).wait()
            pltpu.make_async_copy(dl_hbm.at[c, s, pl.ds(0, EDGE_K)], dbuf,
                                  semi).wait()

        def fire_gather(sbuf, rows, semg):
            pltpu.async_copy(p_hbm.at[sbuf], rows, semg)

        def gather_wait(rows, semg):
            pltpu.make_async_copy(p_hbm.at[pl.ds(0, EDGE_K)], rows, semg).wait()

        def scatter_wait(rows, sems):
            pltpu.make_async_copy(p_hbm.at[pl.ds(0, EDGE_K)], rows, sems).wait()

        def macro(m, drain, cur, nxt):
            (sbuf_c, dbuf_c, dloc_c, rows_c, semi_c, semg_c, sems_c) = cur
            (sbuf_n, dbuf_n, dloc_n, rows_n, semi_n, semg_n, sems_n) = nxt
            idx_wait(sbuf_n, dbuf_n, semi_n)
            if drain:
                scatter_wait(rows_n, sems_n)
            fire_gather(sbuf_n, rows_n, semg_n)
            for kk in range(EDGE_K // 16):
                dloc_c[pl.ds(kk * 16, 16)] = dbuf_c[pl.ds(kk * 16, 16)]
            gather_wait(rows_c, semg_c)
            idx_load(m + 2, sbuf_c, dbuf_c, semi_c)
            pltpu.async_copy(rows_c, acc.at[dloc_c], sems_c, add=True)

        bufs0 = (sbuf0, dbuf0, dloc0, rows0, semi0, semg0, sems0)
        bufs1 = (sbuf1, dbuf1, dloc1, rows1, semi1, semg1, sems1)

        idx_load(jnp.int32(0), sbuf0, dbuf0, semi0)
        idx_load(jnp.int32(1), sbuf1, dbuf1, semi1)
        idx_wait(sbuf0, dbuf0, semi0)
        fire_gather(sbuf0, rows0, semg0)

        macro(jnp.int32(0), False, bufs0, bufs1)
        macro(jnp.int32(1), True, bufs1, bufs0)

        def pair(i, _):
            m = 2 * i
            macro(m, True, bufs0, bufs1)
            macro(m + 1, True, bufs1, bufs0)
            return 0

        lax.fori_loop(1, (trips - 1) // 2, pair, 0)

        scatter_wait(rows1, sems1)
        gather_wait(rows0, semg0)
        pltpu.async_copy(rows0, acc.at[dbuf0], sems0, add=True).wait()
        idx_wait(sbuf1, dbuf1, semi1)
        plsc.subcore_barrier()

        @pl.when(s < NS - 1)
        def _():
            pltpu.sync_copy(
                acc.at[pl.ds(s * ROWS_PT, ROWS_PT)],
                out_hbm.at[pl.ds(base + s * ROWS_PT, ROWS_PT)],
            )

        @pl.when(s == NS - 1)
        def _():
            pltpu.sync_copy(
                acc.at[pl.ds((NS - 1) * ROWS_PT, ROWS_LAST)],
                out_hbm.at[pl.ds(base + (NS - 1) * ROWS_PT, ROWS_LAST)],
            )

    return k(p, plist, dlist, cnts)


def _pool_segsum(sarr, batch):
    @functools.partial(
        pl.kernel,
        out_type=jax.ShapeDtypeStruct((NC * NG, D), jnp.float32),
        mesh=_sc_mesh(),
        compiler_params=pltpu.CompilerParams(use_tc_tiling_on_sc=False),
        scratch_types=[
            pltpu.VMEM((POOL_K,), jnp.int32),
            pltpu.VMEM((POOL_TAIL,), jnp.int32),
            pltpu.VMEM((POOL_K, D), jnp.float32),
            pltpu.VMEM((GPT, D), jnp.float32),
            pltpu.VMEM_SHARED((NG, D), jnp.float32),
        ],
    )
    def k(s_hbm, b_hbm, out_hbm, bbuf, bbuf_t, rows, zbuf, acc):
        c = lax.axis_index("c")
        s = lax.axis_index("s")

        _zero_vmem(zbuf, GPT)
        pltpu.sync_copy(zbuf, acc.at[pl.ds(s * GPT, GPT)])
        plsc.subcore_barrier()

        rbase = c * HALF + s * ROWS_PT
        nfull = jnp.where(s < NS - 1, POOL_FULL, POOL_LAST_FULL)

        def step(kk, _):
            rb = rbase + kk * POOL_K
            pltpu.sync_copy(s_hbm.at[pl.ds(rb, POOL_K)], rows)
            pltpu.sync_copy(b_hbm.at[pl.ds(rb, POOL_K)], bbuf)
            pltpu.sync_copy(rows, acc.at[bbuf], add=True)
            return 0

        lax.fori_loop(0, nfull, step, 0)

        @pl.when(s == NS - 1)
        def _():
            rb = rbase + POOL_LAST_FULL * POOL_K
            pltpu.sync_copy(s_hbm.at[pl.ds(rb, POOL_TAIL)], rows.at[pl.ds(0, POOL_TAIL)])
            pltpu.sync_copy(b_hbm.at[pl.ds(rb, POOL_TAIL)], bbuf_t)
            pltpu.sync_copy(rows.at[pl.ds(0, POOL_TAIL)], acc.at[bbuf_t], add=True)

        plsc.subcore_barrier()
        pltpu.sync_copy(
            acc.at[pl.ds(s * GPT, GPT)],
            out_hbm.at[pl.ds(c * NG + s * GPT, GPT)],
        )

    return k(sarr, batch)


BR = 2000


def _mm_in(x, w1):
    def body(x_ref, w_ref, o_ref):
        o_ref[...] = jnp.dot(x_ref[...], w_ref[...], preferred_element_type=jnp.float32)

    return pl.pallas_call(
        body,
        grid=(N // BR,),
        in_specs=[
            pl.BlockSpec((BR, D_IN), lambda i: (i, 0)),
            pl.BlockSpec((D_IN, D), lambda i: (0, 0)),
        ],
        out_specs=pl.BlockSpec((BR, D), lambda i: (i, 0)),
        out_shape=jax.ShapeDtypeStruct((N, D), jnp.float32),
    )(x, w1)


def _layer_mid(p, agg, part, eps, b1, w2, b2, wnext, jkw):
    def body(eps_ref, p_ref, a_ref, pt_ref, b1_ref, w2_ref, b2_ref, wn_ref,
             jk_ref, pn_ref, po_ref):
        z = (1.0 + eps_ref[0]) * p_ref[...] + a_ref[...] + b1_ref[...]
        r = jnp.maximum(z, 0.0)
        h = jnp.maximum(
            jnp.dot(r, w2_ref[...], preferred_element_type=jnp.float32) + b2_ref[...],
            0.0,
        )
        pn_ref[...] = jnp.dot(h, wn_ref[...], preferred_element_type=jnp.float32)
        po_ref[...] = pt_ref[...] + jnp.dot(
            h, jk_ref[...], preferred_element_type=jnp.float32
        )

    full = lambda shape: pl.BlockSpec(shape, lambda i: tuple(0 for _ in shape))
    return pl.pallas_call(
        body,
        grid=(N // BR,),
        in_specs=[
            pl.BlockSpec(memory_space=pltpu.SMEM),
            pl.BlockSpec((BR, D), lambda i: (i, 0)),
            pl.BlockSpec((BR, D), lambda i: (i, 0)),
            pl.BlockSpec((BR, D), lambda i: (i, 0)),
            full((1, D)),
            full((D, D)),
            full((1, D)),
            full((D, D)),
            full((D, D)),
        ],
        out_specs=[
            pl.BlockSpec((BR, D), lambda i: (i, 0)),
            pl.BlockSpec((BR, D), lambda i: (i, 0)),
        ],
        out_shape=[
            jax.ShapeDtypeStruct((N, D), jnp.float32),
            jax.ShapeDtypeStruct((N, D), jnp.float32),
        ],
    )(eps.reshape(1), p, agg, part, b1.reshape(1, D), w2, b2.reshape(1, D), wnext, jkw)


def _layer_first(p, agg, eps, b1, w2, b2, wnext, jkw):
    def body(eps_ref, p_ref, a_ref, b1_ref, w2_ref, b2_ref, wn_ref, jk_ref,
             pn_ref, po_ref):
        z = (1.0 + eps_ref[0]) * p_ref[...] + a_ref[...] + b1_ref[...]
        r = jnp.maximum(z, 0.0)
        h = jnp.maximum(
            jnp.dot(r, w2_ref[...], preferred_element_type=jnp.float32) + b2_ref[...],
            0.0,
        )
        pn_ref[...] = jnp.dot(h, wn_ref[...], preferred_element_type=jnp.float32)
        po_ref[...] = jnp.dot(h, jk_ref[...], preferred_element_type=jnp.float32)

    full = lambda shape: pl.BlockSpec(shape, lambda i: tuple(0 for _ in shape))
    return pl.pallas_call(
        body,
        grid=(N // BR,),
        in_specs=[
            pl.BlockSpec(memory_space=pltpu.SMEM),
            pl.BlockSpec((BR, D), lambda i: (i, 0)),
            pl.BlockSpec((BR, D), lambda i: (i, 0)),
            full((1, D)),
            full((D, D)),
            full((1, D)),
            full((D, D)),
            full((D, D)),
        ],
        out_specs=[
            pl.BlockSpec((BR, D), lambda i: (i, 0)),
            pl.BlockSpec((BR, D), lambda i: (i, 0)),
        ],
        out_shape=[
            jax.ShapeDtypeStruct((N, D), jnp.float32),
            jax.ShapeDtypeStruct((N, D), jnp.float32),
        ],
    )(eps.reshape(1), p, agg, b1.reshape(1, D), w2, b2.reshape(1, D), wnext, jkw)


def _layer_last(p, agg, part, eps, b1, w2, b2, jkw, jkb):
    def body(eps_ref, p_ref, a_ref, pt_ref, b1_ref, w2_ref, b2_ref, jk_ref,
             jkb_ref, so_ref):
        z = (1.0 + eps_ref[0]) * p_ref[...] + a_ref[...] + b1_ref[...]
        r = jnp.maximum(z, 0.0)
        h = jnp.maximum(
            jnp.dot(r, w2_ref[...], preferred_element_type=jnp.float32) + b2_ref[...],
            0.0,
        )
        so_ref[...] = (
            pt_ref[...]
            + jnp.dot(h, jk_ref[...], preferred_element_type=jnp.float32)
            + jkb_ref[...]
        )

    full = lambda shape: pl.BlockSpec(shape, lambda i: tuple(0 for _ in shape))
    return pl.pallas_call(
        body,
        grid=(N // BR,),
        in_specs=[
            pl.BlockSpec(memory_space=pltpu.SMEM),
            pl.BlockSpec((BR, D), lambda i: (i, 0)),
            pl.BlockSpec((BR, D), lambda i: (i, 0)),
            pl.BlockSpec((BR, D), lambda i: (i, 0)),
            full((1, D)),
            full((D, D)),
            full((1, D)),
            full((D, D)),
            full((1, D)),
        ],
        out_specs=pl.BlockSpec((BR, D), lambda i: (i, 0)),
        out_shape=jax.ShapeDtypeStruct((N, D), jnp.float32),
    )(eps.reshape(1), p, agg, part, b1.reshape(1, D), w2, b2.reshape(1, D), jkw,
      jkb.reshape(1, D))


def _head(gparts, ffn_w1, ffn_b1, bn_g, bn_b, bn_m, bn_v, ffn_w2, ffn_b2,
          out_w, out_b):
    def body(gp_ref, w1_ref, b1_ref, g_ref, bb_ref, m_ref, v_ref, w2_ref,
             b2_ref, ow_ref, ob_ref, o_ref):
        g = gp_ref[:NG, :] + gp_ref[NG:, :]
        f = jnp.dot(g, w1_ref[...], preferred_element_type=jnp.float32) + b1_ref[...]
        scale = g_ref[...] * lax.rsqrt(v_ref[...] + 1e-5)
        f = (f - m_ref[...]) * scale + bb_ref[...]
        f = jnp.maximum(f, 0.0)
        f = jnp.dot(f, w2_ref[...], preferred_element_type=jnp.float32) + b2_ref[...]
        o_ref[...] = (
            jnp.dot(f, ow_ref[...], preferred_element_type=jnp.float32) + ob_ref[...]
        )

    full = lambda shape: pl.BlockSpec(shape, lambda: tuple(0 for _ in shape))
    return pl.pallas_call(
        body,
        in_specs=[
            full((NC * NG, D)),
            full((D, D)),
            full((1, D)),
            full((1, D)),
            full((1, D)),
            full((1, D)),
            full((1, D)),
            full((D, D)),
            full((1, D)),
            full((D, N_OUT)),
            full((1, N_OUT)),
        ],
        out_specs=full((NG, N_OUT)),
        out_shape=jax.ShapeDtypeStruct((NG, N_OUT), jnp.float32),
    )(gparts, ffn_w1, ffn_b1.reshape(1, D), bn_g.reshape(1, D),
      bn_b.reshape(1, D), bn_m.reshape(1, D), bn_v.reshape(1, D), ffn_w2,
      ffn_b2.reshape(1, D), out_w, out_b.reshape(1, N_OUT))


def kernel(x, edge_index, batch, eps0, l0_W1, l0_b1, l0_W2, l0_b2, eps1,
           l1_W1, l1_b1, l1_W2, l1_b2, eps2, l2_W1, l2_b1, l2_W2, l2_b2,
           jk_W, jk_b, ffn_W1, ffn_b1, bn_g, bn_b, bn_m, bn_v, ffn_W2,
           ffn_b2, out_W, out_b):
    src = edge_index[0]
    dst = edge_index[1]
    jkw0, jkw1, jkw2 = jk_W[:D], jk_W[D:2 * D], jk_W[2 * D:]

    plist, dlist, cnts = _compact_edges(src, dst)
    p0 = _mm_in(x, l0_W1)
    a0 = _edge_segsum(p0, plist, dlist, cnts)
    p1, part = _layer_first(p0, a0, eps0, l0_b1, l0_W2, l0_b2, l1_W1, jkw0)
    a1 = _edge_segsum(p1, plist, dlist, cnts)
    p2, part = _layer_mid(p1, a1, part, eps1, l1_b1, l1_W2, l1_b2, l2_W1, jkw1)
    a2 = _edge_segsum(p2, plist, dlist, cnts)
    sarr = _layer_last(p2, a2, part, eps2, l2_b1, l2_W2, l2_b2, jkw2, jk_b)
    gparts = _pool_segsum(sarr, batch)
    return _head(gparts, ffn_W1, ffn_b1, bn_g, bn_b, bn_m, bn_v, ffn_W2,
                 ffn_b2, out_W, out_b)
